# Initial kernel scaffold; baseline (speedup 1.0000x reference)
import jax
import jax.numpy as jnp
from jax.experimental import pallas as pl


def kernel(chars, emb, W1, b1, W2, b2):
    raise NotImplementedError("write your pallas kernel here")


# speedup vs baseline: 62.8538x; 62.8538x over previous
import functools

import jax
import jax.numpy as jnp
from jax.experimental import pallas as pl

CHAR_VOCAB = 256
CHAR_EMB = 16
WORD_LEN = 20
HIDDEN = 128
OUT_DIM = 64

TOKEN_BLOCK = 512


def _expand_table_kernel(emb_ref, w1_ref, m_ref):
    for p in range(WORD_LEN):
        w1p = w1_ref[p * CHAR_EMB:(p + 1) * CHAR_EMB, :]
        m = jnp.dot(emb_ref[...], w1p, preferred_element_type=jnp.float32)
        m_ref[p] = m.astype(jnp.bfloat16)


def _mlp_kernel(chars_ref, m_ref, b1_ref, w2_ref, b2_ref, out_ref):
    t = chars_ref.shape[0]
    acc = jnp.zeros((t, HIDDEN), dtype=jnp.float32)
    iota = jax.lax.broadcasted_iota(jnp.int32, (t, CHAR_VOCAB), 1)
    for p in range(WORD_LEN):
        idx = chars_ref[:, p:p + 1]
        oh = (iota == idx).astype(jnp.bfloat16)
        acc += jnp.dot(oh, m_ref[p], preferred_element_type=jnp.float32)
    h = jax.nn.relu(acc + b1_ref[...])
    out = jnp.dot(h, w2_ref[...], preferred_element_type=jnp.float32)
    out_ref[...] = out + b2_ref[...]


def kernel(chars, emb, W1, b1, W2, b2):
    b, s, w = chars.shape
    n = b * s
    chars2 = chars.reshape(n, w)

    m_tab = pl.pallas_call(
        _expand_table_kernel,
        out_shape=jax.ShapeDtypeStruct((WORD_LEN, CHAR_VOCAB, HIDDEN),
                                       jnp.bfloat16),
    )(emb, W1)

    grid = (n // TOKEN_BLOCK,)
    out = pl.pallas_call(
        _mlp_kernel,
        grid=grid,
        in_specs=[
            pl.BlockSpec((TOKEN_BLOCK, w), lambda i: (i, 0)),
            pl.BlockSpec((WORD_LEN, CHAR_VOCAB, HIDDEN), lambda i: (0, 0, 0)),
            pl.BlockSpec((1, HIDDEN), lambda i: (0, 0)),
            pl.BlockSpec((HIDDEN, OUT_DIM), lambda i: (0, 0)),
            pl.BlockSpec((1, OUT_DIM), lambda i: (0, 0)),
        ],
        out_specs=pl.BlockSpec((TOKEN_BLOCK, OUT_DIM), lambda i: (i, 0)),
        out_shape=jax.ShapeDtypeStruct((n, OUT_DIM), jnp.float32),
    )(chars2, m_tab, b1.reshape(1, HIDDEN), W2, b2.reshape(1, OUT_DIM))

    return out.reshape(b, s, OUT_DIM)
